# baseline (speedup 1.0000x reference)
import jax
import jax.numpy as jnp
from jax import lax
from jax.experimental import pallas as pl
from jax.experimental.pallas import tpu as pltpu
from jax.experimental.pallas import tpu_sc as plsc

N = 10000
E = 320000
D = 128
DC = 64
DH1 = 256

NC = 2
NS = 16

CH = 128
G = 160
EPT = CH * G
E_PAD = EPT * NS

N_ACC = 10240
RZ = N_ACC // NS
KZ = RZ // CH

_MESH = plsc.VectorSubcoreMesh(core_axis_name="c", subcore_axis_name="s",
                               num_cores=NC, num_subcores=NS)


def _sc_agg_body(table2, srcp2, dstp, zf, outf, src_all, dst_all,
                 rows0, rows1, rows2, rows3, zrow_v, accf, sem_g, sem_s):
    cid = lax.axis_index("c")
    sid = lax.axis_index("s")
    row0 = sid * RZ

    pltpu.sync_copy(srcp2.at[pl.ds((cid * NS + sid) * G, G)], src_all)
    pltpu.sync_copy(dstp.at[pl.ds(sid * G, G)], dst_all)

    pltpu.sync_copy(zf, zrow_v)
    for k in range(KZ):
        pltpu.sync_copy(zrow_v, accf.at[pl.ds(row0 + k * CH, CH)])
    plsc.subcore_barrier()

    def gather(g, buf):
        pltpu.async_copy(table2.at[src_all.at[g]], buf, sem_g)

    def drain_gather(g, buf):
        pltpu.make_async_copy(table2.at[src_all.at[g]], buf, sem_g).wait()

    def scatter(g, buf):
        pltpu.sync_copy(buf, accf.at[dst_all.at[g]], add=True)

    gather(0, rows0)

    def body(i, carry):
        g = 2 * i
        gather(g + 1, rows1)
        drain_gather(g, rows0)
        scatter(g, rows0)

        @pl.when(g + 2 < G)
        def _():
            gather(g + 2, rows0)
        drain_gather(g + 1, rows1)
        scatter(g + 1, rows1)
        return carry

    lax.fori_loop(0, G // 2, body, 0)
    plsc.subcore_barrier()

    obase = cid * N_ACC + row0
    for k in range(KZ):
        pltpu.sync_copy(accf.at[pl.ds(row0 + k * CH, CH)], rows0)
        pltpu.sync_copy(rows0, outf.at[pl.ds(obase + k * CH, CH)])


def _sc_agg(table2, srcp2, dstp, zf):
    scratch = [
        pltpu.VMEM((G, CH), jnp.int32),
        pltpu.VMEM((G, CH), jnp.int32),
        pltpu.VMEM((CH, DC), jnp.float32),
        pltpu.VMEM((CH, DC), jnp.float32),
        pltpu.VMEM((CH, DC), jnp.float32),
        pltpu.VMEM((CH, DC), jnp.float32),
        pltpu.VMEM((CH, DC), jnp.float32),
        pltpu.VMEM_SHARED((N_ACC, DC), jnp.float32),
        pltpu.SemaphoreType.DMA,
        pltpu.SemaphoreType.DMA,
    ]
    return pl.kernel(
        _sc_agg_body,
        out_type=jax.ShapeDtypeStruct((NC * N_ACC, DC), jnp.float32),
        mesh=_MESH, scratch_types=scratch,
        compiler_params=pltpu.CompilerParams(use_tc_tiling_on_sc=False),
    )(table2, srcp2, dstp, zf)


GH = G // 2


def _sc_counts_body(dstp, zc, ones_h, outc, dst_all, ones_v, zc_v,
                    accc):
    cid = lax.axis_index("c")
    sid = lax.axis_index("s")
    row0 = sid * RZ

    pltpu.sync_copy(dstp.at[pl.ds(sid * G + cid * GH, GH)], dst_all)
    pltpu.sync_copy(zc, zc_v)
    for k in range(KZ):
        pltpu.sync_copy(zc_v, accc.at[pl.ds(row0 + k * CH, CH)])
    pltpu.sync_copy(ones_h, ones_v)
    plsc.subcore_barrier()

    def body(g, carry):
        pltpu.sync_copy(ones_v, accc.at[dst_all.at[g]], add=True)
        return carry

    lax.fori_loop(0, GH, body, 0)
    plsc.subcore_barrier()

    obase = cid * N_ACC + row0
    for k in range(KZ):
        pltpu.sync_copy(accc.at[pl.ds(row0 + k * CH, CH)], zc_v)
        pltpu.sync_copy(zc_v, outc.at[pl.ds(obase + k * CH, CH)])


def _sc_counts(dstp, zc, ones_h):
    scratch = [
        pltpu.VMEM((GH, CH), jnp.int32),
        pltpu.VMEM((CH, 16), jnp.float32),
        pltpu.VMEM((CH, 16), jnp.float32),
        pltpu.VMEM_SHARED((N_ACC, 16), jnp.float32),
    ]
    return pl.kernel(
        _sc_counts_body,
        out_type=jax.ShapeDtypeStruct((NC * N_ACC, 16), jnp.float32),
        mesh=_MESH, scratch_types=scratch,
        compiler_params=pltpu.CompilerParams(use_tc_tiling_on_sc=False),
    )(dstp, zc, ones_h)

BLK = 1000
GRID = N // BLK


def _tc1_body(pf, pc, x, wl1t, bl1, wr1t, wl2t, h1_o, f2_o, inv_o):
    p = pf[...]
    s = jnp.concatenate([p[0], p[1]], axis=1)
    c = pc[...]
    cnt = c[0, :, 0:1] + c[1, :, 0:1]
    inv = 1.0 / jnp.maximum(cnt, 1.0)
    mean = s * inv
    h1 = jnp.maximum(
        jnp.dot(mean, wl1t[...], preferred_element_type=jnp.float32)
        + bl1[...]
        + jnp.dot(x[...], wr1t[...], preferred_element_type=jnp.float32),
        0.0)
    h1_o[...] = h1
    f2_o[...] = jnp.dot(h1, wl2t[...], preferred_element_type=jnp.float32)
    inv_o[...] = inv


def _tc1(pf, pc, x, wl1t, bl1, wr1t, wl2t):
    return pl.pallas_call(
        _tc1_body,
        grid=(GRID,),
        in_specs=[
            pl.BlockSpec((NC, BLK, DC), lambda i: (0, i, 0)),
            pl.BlockSpec((NC, BLK, 16), lambda i: (0, i, 0)),
            pl.BlockSpec((BLK, D), lambda i: (i, 0)),
            pl.BlockSpec((D, DH1), lambda i: (0, 0)),
            pl.BlockSpec((1, DH1), lambda i: (0, 0)),
            pl.BlockSpec((D, DH1), lambda i: (0, 0)),
            pl.BlockSpec((DH1, D), lambda i: (0, 0)),
        ],
        out_specs=[
            pl.BlockSpec((BLK, DH1), lambda i: (i, 0)),
            pl.BlockSpec((BLK, D), lambda i: (i, 0)),
            pl.BlockSpec((BLK, 1), lambda i: (i, 0)),
        ],
        out_shape=[
            jax.ShapeDtypeStruct((N, DH1), jnp.float32),
            jax.ShapeDtypeStruct((N, D), jnp.float32),
            jax.ShapeDtypeStruct((N, 1), jnp.float32),
        ],
    )(pf, pc, x, wl1t, bl1, wr1t, wl2t)


def _tc2_body(pf, inv, h1, bl2, wr2t, wl3t, h2_o, f3_o):
    p = pf[...]
    s = jnp.concatenate([p[0], p[1]], axis=1)
    h2 = jnp.maximum(
        s * inv[...] + bl2[...]
        + jnp.dot(h1[...], wr2t[...], preferred_element_type=jnp.float32),
        0.0)
    h2_o[...] = h2
    f3_o[...] = jnp.dot(h2, wl3t[...], preferred_element_type=jnp.float32)


def _tc2(pf, inv, h1, bl2, wr2t, wl3t):
    return pl.pallas_call(
        _tc2_body,
        grid=(GRID,),
        in_specs=[
            pl.BlockSpec((NC, BLK, DC), lambda i: (0, i, 0)),
            pl.BlockSpec((BLK, 1), lambda i: (i, 0)),
            pl.BlockSpec((BLK, DH1), lambda i: (i, 0)),
            pl.BlockSpec((1, D), lambda i: (0, 0)),
            pl.BlockSpec((DH1, D), lambda i: (0, 0)),
            pl.BlockSpec((D, D), lambda i: (0, 0)),
        ],
        out_specs=[
            pl.BlockSpec((BLK, D), lambda i: (i, 0)),
            pl.BlockSpec((BLK, D), lambda i: (i, 0)),
        ],
        out_shape=[
            jax.ShapeDtypeStruct((N, D), jnp.float32),
            jax.ShapeDtypeStruct((N, D), jnp.float32),
        ],
    )(pf, inv, h1, bl2, wr2t, wl3t)


def _tc3_body(pf, inv, h1, h2, bl3, wr3t, out_o):
    p = pf[...]
    s = jnp.concatenate([p[0], p[1]], axis=1)
    h3 = jnp.maximum(
        s * inv[...] + bl3[...]
        + jnp.dot(h2[...], wr3t[...], preferred_element_type=jnp.float32),
        0.0)
    out_o[...] = jnp.concatenate([h1[...], h2[...], h3], axis=1)


def _tc3(pf, inv, h1, h2, bl3, wr3t):
    return pl.pallas_call(
        _tc3_body,
        grid=(GRID,),
        in_specs=[
            pl.BlockSpec((NC, BLK, DC), lambda i: (0, i, 0)),
            pl.BlockSpec((BLK, 1), lambda i: (i, 0)),
            pl.BlockSpec((BLK, DH1), lambda i: (i, 0)),
            pl.BlockSpec((BLK, D), lambda i: (i, 0)),
            pl.BlockSpec((1, D), lambda i: (0, 0)),
            pl.BlockSpec((D, D), lambda i: (0, 0)),
        ],
        out_specs=pl.BlockSpec((BLK, DH1 + 2 * D), lambda i: (i, 0)),
        out_shape=jax.ShapeDtypeStruct((N, DH1 + 2 * D), jnp.float32),
    )(pf, inv, h1, h2, bl3, wr3t)


def kernel(x, edge_index, Wl1, bl1, Wr1, Wl2, bl2, Wr2, Wl3, bl3, Wr3):
    src = edge_index[0].astype(jnp.int32)
    dst = edge_index[1].astype(jnp.int32)
    pad = E_PAD - E
    srcp = jnp.concatenate([src, jnp.zeros((pad,), jnp.int32)])
    dstp = jnp.concatenate([dst, jnp.full((pad,), N, jnp.int32)])
    srcp2 = jnp.concatenate([2 * srcp, 2 * srcp + 1]).reshape(
        NC * NS * G, CH)
    dstp = dstp.reshape(NS * G, CH)

    zf = jnp.zeros((CH, DC), jnp.float32)
    zc = jnp.zeros((CH, 16), jnp.float32)
    ones_h = jnp.ones((CH, 16), jnp.float32)

    wl1t = Wl1.T
    wr1t = Wr1.T
    wl2t = Wl2.T
    wr2t = Wr2.T
    wl3t = Wl3.T
    wr3t = Wr3.T
    bl1r = bl1.reshape(1, DH1)
    bl2r = bl2.reshape(1, D)
    bl3r = bl3.reshape(1, D)

    pc1 = _sc_counts(dstp, zc, ones_h)
    pf1 = _sc_agg(x.reshape(2 * N, DC), srcp2, dstp, zf)
    h1, f2, inv = _tc1(pf1.reshape(NC, N_ACC, DC), pc1.reshape(NC, N_ACC, 16),
                       x, wl1t, bl1r, wr1t, wl2t)

    pf2 = _sc_agg(f2.reshape(2 * N, DC), srcp2, dstp, zf)
    h2, f3 = _tc2(pf2.reshape(NC, N_ACC, DC), inv, h1, bl2r, wr2t, wl3t)

    pf3 = _sc_agg(f3.reshape(2 * N, DC), srcp2, dstp, zf)
    out = _tc3(pf3.reshape(NC, N_ACC, DC), inv, h1, h2, bl3r, wr3t)
    return out


# speedup vs baseline: 2.6254x; 2.6254x over previous
import jax
import jax.numpy as jnp
from jax import lax
from jax.experimental import pallas as pl
from jax.experimental.pallas import tpu as pltpu
from jax.experimental.pallas import tpu_sc as plsc

N = 10000
E = 320000
D = 128
DC = 64
DH1 = 256

NC = 2
NS = 16

CH = 128
G = 160
EPT = CH * G
E_PAD = EPT * NS

N_ACC = 10240
RZ = N_ACC // NS
KZ = RZ // CH

_MESH = plsc.VectorSubcoreMesh(core_axis_name="c", subcore_axis_name="s",
                               num_cores=NC, num_subcores=NS)


def _sc_agg_body(table2, srcp2, dstp, zf, outf, src_all, dst_all,
                 rows0, rows1, rows2, rows3, zrow_v, accf, sem_g, sem_s):
    cid = lax.axis_index("c")
    sid = lax.axis_index("s")
    row0 = sid * RZ

    pltpu.sync_copy(srcp2.at[pl.ds((cid * NS + sid) * G, G)], src_all)
    pltpu.sync_copy(dstp.at[pl.ds(sid * G, G)], dst_all)

    pltpu.sync_copy(zf, zrow_v)
    for k in range(KZ):
        pltpu.sync_copy(zrow_v, accf.at[pl.ds(row0 + k * CH, CH)])
    plsc.subcore_barrier()

    def gather(g, buf):
        pltpu.async_copy(table2.at[src_all.at[g]], buf, sem_g)

    def drain_gather(g, buf):
        pltpu.make_async_copy(table2.at[src_all.at[g]], buf, sem_g).wait()

    def scatter(g, buf):
        pltpu.sync_copy(buf, accf.at[dst_all.at[g]], add=True)

    gather(0, rows0)

    def body(i, carry):
        g = 2 * i
        gather(g + 1, rows1)
        drain_gather(g, rows0)
        scatter(g, rows0)

        @pl.when(g + 2 < G)
        def _():
            gather(g + 2, rows0)
        drain_gather(g + 1, rows1)
        scatter(g + 1, rows1)
        return carry

    lax.fori_loop(0, G // 2, body, 0)
    plsc.subcore_barrier()

    obase = cid * N_ACC + row0
    for k in range(KZ):
        pltpu.sync_copy(accf.at[pl.ds(row0 + k * CH, CH)], rows0)
        pltpu.sync_copy(rows0, outf.at[pl.ds(obase + k * CH, CH)])


def _sc_agg(table2, srcp2, dstp, zf):
    scratch = [
        pltpu.VMEM((G, CH), jnp.int32),
        pltpu.VMEM((G, CH), jnp.int32),
        pltpu.VMEM((CH, DC), jnp.float32),
        pltpu.VMEM((CH, DC), jnp.float32),
        pltpu.VMEM((CH, DC), jnp.float32),
        pltpu.VMEM((CH, DC), jnp.float32),
        pltpu.VMEM((CH, DC), jnp.float32),
        pltpu.VMEM_SHARED((N_ACC, DC), jnp.float32),
        pltpu.SemaphoreType.DMA,
        pltpu.SemaphoreType.DMA,
    ]
    return pl.kernel(
        _sc_agg_body,
        out_type=jax.ShapeDtypeStruct((NC * N_ACC, DC), jnp.float32),
        mesh=_MESH, scratch_types=scratch,
        compiler_params=pltpu.CompilerParams(use_tc_tiling_on_sc=False),
    )(table2, srcp2, dstp, zf)


GH = G // 2


def _sc_counts_body(dstp, zc, ones_h, outc, dst_all, ones_v, zc_v,
                    accc):
    cid = lax.axis_index("c")
    sid = lax.axis_index("s")
    row0 = sid * RZ

    pltpu.sync_copy(dstp.at[pl.ds(sid * G + cid * GH, GH)], dst_all)
    pltpu.sync_copy(zc, zc_v)
    for k in range(KZ):
        pltpu.sync_copy(zc_v, accc.at[pl.ds(row0 + k * CH, CH)])
    pltpu.sync_copy(ones_h, ones_v)
    plsc.subcore_barrier()

    def body(g, carry):
        pltpu.sync_copy(ones_v, accc.at[dst_all.at[g]], add=True)
        return carry

    lax.fori_loop(0, GH, body, 0)
    plsc.subcore_barrier()

    obase = cid * N_ACC + row0
    for k in range(KZ):
        pltpu.sync_copy(accc.at[pl.ds(row0 + k * CH, CH)], zc_v)
        pltpu.sync_copy(zc_v, outc.at[pl.ds(obase + k * CH, CH)])


def _sc_counts(dstp, zc, ones_h):
    scratch = [
        pltpu.VMEM((GH, CH), jnp.int32),
        pltpu.VMEM((CH, 16), jnp.float32),
        pltpu.VMEM((CH, 16), jnp.float32),
        pltpu.VMEM_SHARED((N_ACC, 16), jnp.float32),
    ]
    return pl.kernel(
        _sc_counts_body,
        out_type=jax.ShapeDtypeStruct((NC * N_ACC, 16), jnp.float32),
        mesh=_MESH, scratch_types=scratch,
        compiler_params=pltpu.CompilerParams(use_tc_tiling_on_sc=False),
    )(dstp, zc, ones_h)

BLK = 1000
GRID = N // BLK


def _tc1_body(pf, pc, x, wl1t, bl1, wr1t, wl2t, h1_o, f2_o, inv_o):
    p = pf[...]
    s = jnp.concatenate([p[0], p[1]], axis=1)
    c = pc[...]
    cnt = c[0, :, 0:1] + c[1, :, 0:1]
    inv = 1.0 / jnp.maximum(cnt, 1.0)
    mean = s * inv
    h1 = jnp.maximum(
        jnp.dot(mean, wl1t[...], preferred_element_type=jnp.float32)
        + bl1[...]
        + jnp.dot(x[...], wr1t[...], preferred_element_type=jnp.float32),
        0.0)
    h1_o[...] = h1
    f2_o[...] = jnp.dot(h1, wl2t[...], preferred_element_type=jnp.float32)
    inv_o[...] = inv


def _tc1(pf, pc, x, wl1t, bl1, wr1t, wl2t):
    return pl.pallas_call(
        _tc1_body,
        grid=(GRID,),
        in_specs=[
            pl.BlockSpec((NC, BLK, DC), lambda i: (0, i, 0)),
            pl.BlockSpec((NC, BLK, 16), lambda i: (0, i, 0)),
            pl.BlockSpec((BLK, D), lambda i: (i, 0)),
            pl.BlockSpec((D, DH1), lambda i: (0, 0)),
            pl.BlockSpec((1, DH1), lambda i: (0, 0)),
            pl.BlockSpec((D, DH1), lambda i: (0, 0)),
            pl.BlockSpec((DH1, D), lambda i: (0, 0)),
        ],
        out_specs=[
            pl.BlockSpec((BLK, DH1), lambda i: (i, 0)),
            pl.BlockSpec((BLK, D), lambda i: (i, 0)),
            pl.BlockSpec((BLK, 1), lambda i: (i, 0)),
        ],
        out_shape=[
            jax.ShapeDtypeStruct((N, DH1), jnp.float32),
            jax.ShapeDtypeStruct((N, D), jnp.float32),
            jax.ShapeDtypeStruct((N, 1), jnp.float32),
        ],
    )(pf, pc, x, wl1t, bl1, wr1t, wl2t)


def _tc2_body(pf, inv, h1, bl2, wr2t, wl3t, h2_o, f3_o):
    p = pf[...]
    s = jnp.concatenate([p[0], p[1]], axis=1)
    h2 = jnp.maximum(
        s * inv[...] + bl2[...]
        + jnp.dot(h1[...], wr2t[...], preferred_element_type=jnp.float32),
        0.0)
    h2_o[...] = h2
    f3_o[...] = jnp.dot(h2, wl3t[...], preferred_element_type=jnp.float32)


def _tc2(pf, inv, h1, bl2, wr2t, wl3t):
    return pl.pallas_call(
        _tc2_body,
        grid=(GRID,),
        in_specs=[
            pl.BlockSpec((NC, BLK, DC), lambda i: (0, i, 0)),
            pl.BlockSpec((BLK, 1), lambda i: (i, 0)),
            pl.BlockSpec((BLK, DH1), lambda i: (i, 0)),
            pl.BlockSpec((1, D), lambda i: (0, 0)),
            pl.BlockSpec((DH1, D), lambda i: (0, 0)),
            pl.BlockSpec((D, D), lambda i: (0, 0)),
        ],
        out_specs=[
            pl.BlockSpec((BLK, D), lambda i: (i, 0)),
            pl.BlockSpec((BLK, D), lambda i: (i, 0)),
        ],
        out_shape=[
            jax.ShapeDtypeStruct((N, D), jnp.float32),
            jax.ShapeDtypeStruct((N, D), jnp.float32),
        ],
    )(pf, inv, h1, bl2, wr2t, wl3t)


def _tc3_body(pf, inv, h1, h2, bl3, wr3t, out_o):
    p = pf[...]
    s = jnp.concatenate([p[0], p[1]], axis=1)
    h3 = jnp.maximum(
        s * inv[...] + bl3[...]
        + jnp.dot(h2[...], wr3t[...], preferred_element_type=jnp.float32),
        0.0)
    out_o[...] = jnp.concatenate([h1[...], h2[...], h3], axis=1)


def _tc3(pf, inv, h1, h2, bl3, wr3t):
    return pl.pallas_call(
        _tc3_body,
        grid=(GRID,),
        in_specs=[
            pl.BlockSpec((NC, BLK, DC), lambda i: (0, i, 0)),
            pl.BlockSpec((BLK, 1), lambda i: (i, 0)),
            pl.BlockSpec((BLK, DH1), lambda i: (i, 0)),
            pl.BlockSpec((BLK, D), lambda i: (i, 0)),
            pl.BlockSpec((1, D), lambda i: (0, 0)),
            pl.BlockSpec((D, D), lambda i: (0, 0)),
        ],
        out_specs=pl.BlockSpec((BLK, DH1 + 2 * D), lambda i: (i, 0)),
        out_shape=jax.ShapeDtypeStruct((N, DH1 + 2 * D), jnp.float32),
    )(pf, inv, h1, h2, bl3, wr3t)


def kernel(x, edge_index, Wl1, bl1, Wr1, Wl2, bl2, Wr2, Wl3, bl3, Wr3):
    src = edge_index[0].astype(jnp.int32)
    dst = edge_index[1].astype(jnp.int32)
    pad = E_PAD - E
    ar = jnp.arange(pad, dtype=jnp.int32)
    srcp = jnp.concatenate([src, ar % N])
    dstp = jnp.concatenate([dst, N + ar % (N_ACC - N)])
    srcp2 = jnp.concatenate([2 * srcp, 2 * srcp + 1]).reshape(
        NC * NS * G, CH)
    dstp = dstp.reshape(NS * G, CH)

    zf = jnp.zeros((CH, DC), jnp.float32)
    zc = jnp.zeros((CH, 16), jnp.float32)
    ones_h = jnp.ones((CH, 16), jnp.float32)

    wl1t = Wl1.T
    wr1t = Wr1.T
    wl2t = Wl2.T
    wr2t = Wr2.T
    wl3t = Wl3.T
    wr3t = Wr3.T
    bl1r = bl1.reshape(1, DH1)
    bl2r = bl2.reshape(1, D)
    bl3r = bl3.reshape(1, D)

    pc1 = _sc_counts(dstp, zc, ones_h)
    pf1 = _sc_agg(x.reshape(2 * N, DC), srcp2, dstp, zf)
    h1, f2, inv = _tc1(pf1.reshape(NC, N_ACC, DC), pc1.reshape(NC, N_ACC, 16),
                       x, wl1t, bl1r, wr1t, wl2t)

    pf2 = _sc_agg(f2.reshape(2 * N, DC), srcp2, dstp, zf)
    h2, f3 = _tc2(pf2.reshape(NC, N_ACC, DC), inv, h1, bl2r, wr2t, wl3t)

    pf3 = _sc_agg(f3.reshape(2 * N, DC), srcp2, dstp, zf)
    out = _tc3(pf3.reshape(NC, N_ACC, DC), inv, h1, h2, bl3r, wr3t)
    return out


# speedup vs baseline: 3.2509x; 1.2383x over previous
import jax
import jax.numpy as jnp
from jax import lax
from jax.experimental import pallas as pl
from jax.experimental.pallas import tpu as pltpu
from jax.experimental.pallas import tpu_sc as plsc

N = 10000
E = 320000
D = 128
DC = 64
DH1 = 256

NC = 2
NS = 16

CH = 128
G = 160
EPT = CH * G
E_PAD = EPT * NS

N_ACC = 10240
RZ = N_ACC // NS
KZ = RZ // CH

_MESH = plsc.VectorSubcoreMesh(core_axis_name="c", subcore_axis_name="s",
                               num_cores=NC, num_subcores=NS)


def _sc_agg_body(table2, srcp2, dstp, zf, outf, src_all, dst_all,
                 rows0, rows1, rows2, rows3, zrow_v, accf, sem_g, sem_s):
    cid = lax.axis_index("c")
    sid = lax.axis_index("s")
    row0 = sid * RZ

    pltpu.sync_copy(srcp2.at[pl.ds((cid * NS + sid) * G, G)], src_all)
    pltpu.sync_copy(dstp.at[pl.ds(sid * G, G)], dst_all)

    pltpu.sync_copy(zf, zrow_v)
    for k in range(KZ):
        pltpu.sync_copy(zrow_v, accf.at[pl.ds(row0 + k * CH, CH)])
    plsc.subcore_barrier()

    def gather(g, buf):
        pltpu.async_copy(table2.at[src_all.at[g]], buf, sem_g)

    def drain_gather(g, buf):
        pltpu.make_async_copy(table2.at[src_all.at[g]], buf, sem_g).wait()

    def scatter(g, buf):
        pltpu.sync_copy(buf, accf.at[dst_all.at[g]], add=True)

    rows = (rows0, rows1, rows2, rows3)
    gather(0, rows[0])
    gather(1, rows[1])
    gather(2, rows[2])

    def body(i, carry):
        for b in range(4):
            g = 4 * i + b
            buf = rows[b]
            drain_gather(g, buf)

            @pl.when(g + 3 < G)
            def _():
                gather(g + 3, rows[(b + 3) % 4])
            scatter(g, buf)
        return carry

    lax.fori_loop(0, G // 4, body, 0)
    plsc.subcore_barrier()

    obase = cid * N_ACC + row0
    for k in range(KZ):
        pltpu.sync_copy(accf.at[pl.ds(row0 + k * CH, CH)], rows0)
        pltpu.sync_copy(rows0, outf.at[pl.ds(obase + k * CH, CH)])


def _sc_agg(table2, srcp2, dstp, zf):
    scratch = [
        pltpu.VMEM((G, CH), jnp.int32),
        pltpu.VMEM((G, CH), jnp.int32),
        pltpu.VMEM((CH, DC), jnp.float32),
        pltpu.VMEM((CH, DC), jnp.float32),
        pltpu.VMEM((CH, DC), jnp.float32),
        pltpu.VMEM((CH, DC), jnp.float32),
        pltpu.VMEM((CH, DC), jnp.float32),
        pltpu.VMEM_SHARED((N_ACC, DC), jnp.float32),
        pltpu.SemaphoreType.DMA,
        pltpu.SemaphoreType.DMA,
    ]
    return pl.kernel(
        _sc_agg_body,
        out_type=jax.ShapeDtypeStruct((NC * N_ACC, DC), jnp.float32),
        mesh=_MESH, scratch_types=scratch,
        compiler_params=pltpu.CompilerParams(use_tc_tiling_on_sc=False),
    )(table2, srcp2, dstp, zf)


GH = G // 2


def _sc_counts_body(dstp, zc, ones_h, outc, dst_all, ones_v, zc_v,
                    accc):
    cid = lax.axis_index("c")
    sid = lax.axis_index("s")
    row0 = sid * RZ

    pltpu.sync_copy(dstp.at[pl.ds(sid * G + cid * GH, GH)], dst_all)
    pltpu.sync_copy(zc, zc_v)
    for k in range(KZ):
        pltpu.sync_copy(zc_v, accc.at[pl.ds(row0 + k * CH, CH)])
    pltpu.sync_copy(ones_h, ones_v)
    plsc.subcore_barrier()

    def body(g, carry):
        pltpu.sync_copy(ones_v, accc.at[dst_all.at[g]], add=True)
        return carry

    lax.fori_loop(0, GH, body, 0)
    plsc.subcore_barrier()

    obase = cid * N_ACC + row0
    for k in range(KZ):
        pltpu.sync_copy(accc.at[pl.ds(row0 + k * CH, CH)], zc_v)
        pltpu.sync_copy(zc_v, outc.at[pl.ds(obase + k * CH, CH)])


def _sc_counts(dstp, zc, ones_h):
    scratch = [
        pltpu.VMEM((GH, CH), jnp.int32),
        pltpu.VMEM((CH, 16), jnp.float32),
        pltpu.VMEM((CH, 16), jnp.float32),
        pltpu.VMEM_SHARED((N_ACC, 16), jnp.float32),
    ]
    return pl.kernel(
        _sc_counts_body,
        out_type=jax.ShapeDtypeStruct((NC * N_ACC, 16), jnp.float32),
        mesh=_MESH, scratch_types=scratch,
        compiler_params=pltpu.CompilerParams(use_tc_tiling_on_sc=False),
    )(dstp, zc, ones_h)

BLK = 1000
GRID = N // BLK


def _tc1_body(pf, pc, x, wl1t, bl1, wr1t, wl2t, h1_o, f2_o, inv_o):
    p = pf[...]
    s = jnp.concatenate([p[0], p[1]], axis=1)
    c = pc[...]
    cnt = c[0, :, 0:1] + c[1, :, 0:1]
    inv = 1.0 / jnp.maximum(cnt, 1.0)
    mean = s * inv
    h1 = jnp.maximum(
        jnp.dot(mean, wl1t[...], preferred_element_type=jnp.float32)
        + bl1[...]
        + jnp.dot(x[...], wr1t[...], preferred_element_type=jnp.float32),
        0.0)
    h1_o[...] = h1
    f2_o[...] = jnp.dot(h1, wl2t[...], preferred_element_type=jnp.float32)
    inv_o[...] = inv


def _tc1(pf, pc, x, wl1t, bl1, wr1t, wl2t):
    return pl.pallas_call(
        _tc1_body,
        grid=(GRID,),
        in_specs=[
            pl.BlockSpec((NC, BLK, DC), lambda i: (0, i, 0)),
            pl.BlockSpec((NC, BLK, 16), lambda i: (0, i, 0)),
            pl.BlockSpec((BLK, D), lambda i: (i, 0)),
            pl.BlockSpec((D, DH1), lambda i: (0, 0)),
            pl.BlockSpec((1, DH1), lambda i: (0, 0)),
            pl.BlockSpec((D, DH1), lambda i: (0, 0)),
            pl.BlockSpec((DH1, D), lambda i: (0, 0)),
        ],
        out_specs=[
            pl.BlockSpec((BLK, DH1), lambda i: (i, 0)),
            pl.BlockSpec((BLK, D), lambda i: (i, 0)),
            pl.BlockSpec((BLK, 1), lambda i: (i, 0)),
        ],
        out_shape=[
            jax.ShapeDtypeStruct((N, DH1), jnp.float32),
            jax.ShapeDtypeStruct((N, D), jnp.float32),
            jax.ShapeDtypeStruct((N, 1), jnp.float32),
        ],
    )(pf, pc, x, wl1t, bl1, wr1t, wl2t)


def _tc2_body(pf, inv, h1, bl2, wr2t, wl3t, h2_o, f3_o):
    p = pf[...]
    s = jnp.concatenate([p[0], p[1]], axis=1)
    h2 = jnp.maximum(
        s * inv[...] + bl2[...]
        + jnp.dot(h1[...], wr2t[...], preferred_element_type=jnp.float32),
        0.0)
    h2_o[...] = h2
    f3_o[...] = jnp.dot(h2, wl3t[...], preferred_element_type=jnp.float32)


def _tc2(pf, inv, h1, bl2, wr2t, wl3t):
    return pl.pallas_call(
        _tc2_body,
        grid=(GRID,),
        in_specs=[
            pl.BlockSpec((NC, BLK, DC), lambda i: (0, i, 0)),
            pl.BlockSpec((BLK, 1), lambda i: (i, 0)),
            pl.BlockSpec((BLK, DH1), lambda i: (i, 0)),
            pl.BlockSpec((1, D), lambda i: (0, 0)),
            pl.BlockSpec((DH1, D), lambda i: (0, 0)),
            pl.BlockSpec((D, D), lambda i: (0, 0)),
        ],
        out_specs=[
            pl.BlockSpec((BLK, D), lambda i: (i, 0)),
            pl.BlockSpec((BLK, D), lambda i: (i, 0)),
        ],
        out_shape=[
            jax.ShapeDtypeStruct((N, D), jnp.float32),
            jax.ShapeDtypeStruct((N, D), jnp.float32),
        ],
    )(pf, inv, h1, bl2, wr2t, wl3t)


def _tc3_body(pf, inv, h1, h2, bl3, wr3t, out_o):
    p = pf[...]
    s = jnp.concatenate([p[0], p[1]], axis=1)
    h3 = jnp.maximum(
        s * inv[...] + bl3[...]
        + jnp.dot(h2[...], wr3t[...], preferred_element_type=jnp.float32),
        0.0)
    out_o[...] = jnp.concatenate([h1[...], h2[...], h3], axis=1)


def _tc3(pf, inv, h1, h2, bl3, wr3t):
    return pl.pallas_call(
        _tc3_body,
        grid=(GRID,),
        in_specs=[
            pl.BlockSpec((NC, BLK, DC), lambda i: (0, i, 0)),
            pl.BlockSpec((BLK, 1), lambda i: (i, 0)),
            pl.BlockSpec((BLK, DH1), lambda i: (i, 0)),
            pl.BlockSpec((BLK, D), lambda i: (i, 0)),
            pl.BlockSpec((1, D), lambda i: (0, 0)),
            pl.BlockSpec((D, D), lambda i: (0, 0)),
        ],
        out_specs=pl.BlockSpec((BLK, DH1 + 2 * D), lambda i: (i, 0)),
        out_shape=jax.ShapeDtypeStruct((N, DH1 + 2 * D), jnp.float32),
    )(pf, inv, h1, h2, bl3, wr3t)


def kernel(x, edge_index, Wl1, bl1, Wr1, Wl2, bl2, Wr2, Wl3, bl3, Wr3):
    src = edge_index[0].astype(jnp.int32)
    dst = edge_index[1].astype(jnp.int32)
    pad = E_PAD - E
    ar = jnp.arange(pad, dtype=jnp.int32)
    srcp = jnp.concatenate([src, ar % N])
    dstp = jnp.concatenate([dst, N + ar % (N_ACC - N)])
    srcp2 = jnp.concatenate([2 * srcp, 2 * srcp + 1]).reshape(
        NC * NS * G, CH)
    dstp = dstp.reshape(NS * G, CH)

    zf = jnp.zeros((CH, DC), jnp.float32)
    zc = jnp.zeros((CH, 16), jnp.float32)
    ones_h = jnp.ones((CH, 16), jnp.float32)

    wl1t = Wl1.T
    wr1t = Wr1.T
    wl2t = Wl2.T
    wr2t = Wr2.T
    wl3t = Wl3.T
    wr3t = Wr3.T
    bl1r = bl1.reshape(1, DH1)
    bl2r = bl2.reshape(1, D)
    bl3r = bl3.reshape(1, D)

    pc1 = _sc_counts(dstp, zc, ones_h)
    pf1 = _sc_agg(x.reshape(2 * N, DC), srcp2, dstp, zf)
    h1, f2, inv = _tc1(pf1.reshape(NC, N_ACC, DC), pc1.reshape(NC, N_ACC, 16),
                       x, wl1t, bl1r, wr1t, wl2t)

    pf2 = _sc_agg(f2.reshape(2 * N, DC), srcp2, dstp, zf)
    h2, f3 = _tc2(pf2.reshape(NC, N_ACC, DC), inv, h1, bl2r, wr2t, wl3t)

    pf3 = _sc_agg(f3.reshape(2 * N, DC), srcp2, dstp, zf)
    out = _tc3(pf3.reshape(NC, N_ACC, DC), inv, h1, h2, bl3r, wr3t)
    return out


# speedup vs baseline: 3.2511x; 1.0001x over previous
import jax
import jax.numpy as jnp
from jax import lax
from jax.experimental import pallas as pl
from jax.experimental.pallas import tpu as pltpu
from jax.experimental.pallas import tpu_sc as plsc

N = 10000
E = 320000
D = 128
DC = 64
DH1 = 256

NC = 2
NS = 16

CH = 128
G = 160
EPT = CH * G
E_PAD = EPT * NS

N_ACC = 10240
RZ = N_ACC // NS
KZ = RZ // CH

_MESH = plsc.VectorSubcoreMesh(core_axis_name="c", subcore_axis_name="s",
                               num_cores=NC, num_subcores=NS)


def _sc_agg_body(table2, srcp2, dstp, zf, outf, src_all, dst_all,
                 rows0, rows1, rows2, rows3, zrow_v, accf, sem_g, sem_s):
    cid = lax.axis_index("c")
    sid = lax.axis_index("s")
    row0 = sid * RZ

    pltpu.sync_copy(srcp2.at[pl.ds((cid * NS + sid) * G, G)], src_all)
    pltpu.sync_copy(dstp.at[pl.ds(sid * G, G)], dst_all)

    pltpu.sync_copy(zf, zrow_v)
    for k in range(KZ):
        pltpu.sync_copy(zrow_v, accf.at[pl.ds(row0 + k * CH, CH)])
    plsc.subcore_barrier()

    def gather(g, buf):
        pltpu.async_copy(table2.at[src_all.at[g]], buf, sem_g)

    def drain_gather(g, buf):
        pltpu.make_async_copy(table2.at[src_all.at[g]], buf, sem_g).wait()

    def scatter(g, buf):
        pltpu.async_copy(buf, accf.at[dst_all.at[g]], sem_s, add=True)

    def drain_scatter(g, buf):
        pltpu.make_async_copy(buf, accf.at[dst_all.at[g]], sem_s).wait()

    rows = (rows0, rows1, rows2, rows3)
    gather(0, rows[0])
    gather(1, rows[1])
    gather(2, rows[2])

    def body(i, carry):
        for b in range(4):
            g = 4 * i + b
            buf = rows[b]
            drain_gather(g, buf)
            scatter(g, buf)

            @pl.when(g + 3 < G)
            def _():
                @pl.when(g >= 1)
                def _():
                    drain_scatter(g - 1, rows[(b + 3) % 4])
                gather(g + 3, rows[(b + 3) % 4])
        return carry

    lax.fori_loop(0, G // 4, body, 0)
    for g in (G - 4, G - 3, G - 2, G - 1):
        drain_scatter(g, rows[g % 4])
    plsc.subcore_barrier()

    obase = cid * N_ACC + row0
    for k in range(KZ):
        pltpu.sync_copy(accf.at[pl.ds(row0 + k * CH, CH)], rows0)
        pltpu.sync_copy(rows0, outf.at[pl.ds(obase + k * CH, CH)])


def _sc_agg(table2, srcp2, dstp, zf):
    scratch = [
        pltpu.VMEM((G, CH), jnp.int32),
        pltpu.VMEM((G, CH), jnp.int32),
        pltpu.VMEM((CH, DC), jnp.float32),
        pltpu.VMEM((CH, DC), jnp.float32),
        pltpu.VMEM((CH, DC), jnp.float32),
        pltpu.VMEM((CH, DC), jnp.float32),
        pltpu.VMEM((CH, DC), jnp.float32),
        pltpu.VMEM_SHARED((N_ACC, DC), jnp.float32),
        pltpu.SemaphoreType.DMA,
        pltpu.SemaphoreType.DMA,
    ]
    return pl.kernel(
        _sc_agg_body,
        out_type=jax.ShapeDtypeStruct((NC * N_ACC, DC), jnp.float32),
        mesh=_MESH, scratch_types=scratch,
        compiler_params=pltpu.CompilerParams(use_tc_tiling_on_sc=False),
    )(table2, srcp2, dstp, zf)


GH = G // 2


def _sc_counts_body(dstp, zc, ones_h, outc, dst_all, ones_v, zc_v,
                    accc):
    cid = lax.axis_index("c")
    sid = lax.axis_index("s")
    row0 = sid * RZ

    pltpu.sync_copy(dstp.at[pl.ds(sid * G + cid * GH, GH)], dst_all)
    pltpu.sync_copy(zc, zc_v)
    for k in range(KZ):
        pltpu.sync_copy(zc_v, accc.at[pl.ds(row0 + k * CH, CH)])
    pltpu.sync_copy(ones_h, ones_v)
    plsc.subcore_barrier()

    def body(g, carry):
        pltpu.sync_copy(ones_v, accc.at[dst_all.at[g]], add=True)
        return carry

    lax.fori_loop(0, GH, body, 0)
    plsc.subcore_barrier()

    obase = cid * N_ACC + row0
    for k in range(KZ):
        pltpu.sync_copy(accc.at[pl.ds(row0 + k * CH, CH)], zc_v)
        pltpu.sync_copy(zc_v, outc.at[pl.ds(obase + k * CH, CH)])


def _sc_counts(dstp, zc, ones_h):
    scratch = [
        pltpu.VMEM((GH, CH), jnp.int32),
        pltpu.VMEM((CH, 16), jnp.float32),
        pltpu.VMEM((CH, 16), jnp.float32),
        pltpu.VMEM_SHARED((N_ACC, 16), jnp.float32),
    ]
    return pl.kernel(
        _sc_counts_body,
        out_type=jax.ShapeDtypeStruct((NC * N_ACC, 16), jnp.float32),
        mesh=_MESH, scratch_types=scratch,
        compiler_params=pltpu.CompilerParams(use_tc_tiling_on_sc=False),
    )(dstp, zc, ones_h)

BLK = 1000
GRID = N // BLK


def _tc1_body(pf, pc, x, wl1t, bl1, wr1t, wl2t, h1_o, f2_o, inv_o):
    p = pf[...]
    s = jnp.concatenate([p[0], p[1]], axis=1)
    c = pc[...]
    cnt = c[0, :, 0:1] + c[1, :, 0:1]
    inv = 1.0 / jnp.maximum(cnt, 1.0)
    mean = s * inv
    h1 = jnp.maximum(
        jnp.dot(mean, wl1t[...], preferred_element_type=jnp.float32)
        + bl1[...]
        + jnp.dot(x[...], wr1t[...], preferred_element_type=jnp.float32),
        0.0)
    h1_o[...] = h1
    f2_o[...] = jnp.dot(h1, wl2t[...], preferred_element_type=jnp.float32)
    inv_o[...] = inv


def _tc1(pf, pc, x, wl1t, bl1, wr1t, wl2t):
    return pl.pallas_call(
        _tc1_body,
        grid=(GRID,),
        in_specs=[
            pl.BlockSpec((NC, BLK, DC), lambda i: (0, i, 0)),
            pl.BlockSpec((NC, BLK, 16), lambda i: (0, i, 0)),
            pl.BlockSpec((BLK, D), lambda i: (i, 0)),
            pl.BlockSpec((D, DH1), lambda i: (0, 0)),
            pl.BlockSpec((1, DH1), lambda i: (0, 0)),
            pl.BlockSpec((D, DH1), lambda i: (0, 0)),
            pl.BlockSpec((DH1, D), lambda i: (0, 0)),
        ],
        out_specs=[
            pl.BlockSpec((BLK, DH1), lambda i: (i, 0)),
            pl.BlockSpec((BLK, D), lambda i: (i, 0)),
            pl.BlockSpec((BLK, 1), lambda i: (i, 0)),
        ],
        out_shape=[
            jax.ShapeDtypeStruct((N, DH1), jnp.float32),
            jax.ShapeDtypeStruct((N, D), jnp.float32),
            jax.ShapeDtypeStruct((N, 1), jnp.float32),
        ],
    )(pf, pc, x, wl1t, bl1, wr1t, wl2t)


def _tc2_body(pf, inv, h1, bl2, wr2t, wl3t, h2_o, f3_o):
    p = pf[...]
    s = jnp.concatenate([p[0], p[1]], axis=1)
    h2 = jnp.maximum(
        s * inv[...] + bl2[...]
        + jnp.dot(h1[...], wr2t[...], preferred_element_type=jnp.float32),
        0.0)
    h2_o[...] = h2
    f3_o[...] = jnp.dot(h2, wl3t[...], preferred_element_type=jnp.float32)


def _tc2(pf, inv, h1, bl2, wr2t, wl3t):
    return pl.pallas_call(
        _tc2_body,
        grid=(GRID,),
        in_specs=[
            pl.BlockSpec((NC, BLK, DC), lambda i: (0, i, 0)),
            pl.BlockSpec((BLK, 1), lambda i: (i, 0)),
            pl.BlockSpec((BLK, DH1), lambda i: (i, 0)),
            pl.BlockSpec((1, D), lambda i: (0, 0)),
            pl.BlockSpec((DH1, D), lambda i: (0, 0)),
            pl.BlockSpec((D, D), lambda i: (0, 0)),
        ],
        out_specs=[
            pl.BlockSpec((BLK, D), lambda i: (i, 0)),
            pl.BlockSpec((BLK, D), lambda i: (i, 0)),
        ],
        out_shape=[
            jax.ShapeDtypeStruct((N, D), jnp.float32),
            jax.ShapeDtypeStruct((N, D), jnp.float32),
        ],
    )(pf, inv, h1, bl2, wr2t, wl3t)


def _tc3_body(pf, inv, h1, h2, bl3, wr3t, out_o):
    p = pf[...]
    s = jnp.concatenate([p[0], p[1]], axis=1)
    h3 = jnp.maximum(
        s * inv[...] + bl3[...]
        + jnp.dot(h2[...], wr3t[...], preferred_element_type=jnp.float32),
        0.0)
    out_o[...] = jnp.concatenate([h1[...], h2[...], h3], axis=1)


def _tc3(pf, inv, h1, h2, bl3, wr3t):
    return pl.pallas_call(
        _tc3_body,
        grid=(GRID,),
        in_specs=[
            pl.BlockSpec((NC, BLK, DC), lambda i: (0, i, 0)),
            pl.BlockSpec((BLK, 1), lambda i: (i, 0)),
            pl.BlockSpec((BLK, DH1), lambda i: (i, 0)),
            pl.BlockSpec((BLK, D), lambda i: (i, 0)),
            pl.BlockSpec((1, D), lambda i: (0, 0)),
            pl.BlockSpec((D, D), lambda i: (0, 0)),
        ],
        out_specs=pl.BlockSpec((BLK, DH1 + 2 * D), lambda i: (i, 0)),
        out_shape=jax.ShapeDtypeStruct((N, DH1 + 2 * D), jnp.float32),
    )(pf, inv, h1, h2, bl3, wr3t)


def kernel(x, edge_index, Wl1, bl1, Wr1, Wl2, bl2, Wr2, Wl3, bl3, Wr3):
    src = edge_index[0].astype(jnp.int32)
    dst = edge_index[1].astype(jnp.int32)
    pad = E_PAD - E
    ar = jnp.arange(pad, dtype=jnp.int32)
    srcp = jnp.concatenate([src, ar % N])
    dstp = jnp.concatenate([dst, N + ar % (N_ACC - N)])
    srcp2 = jnp.concatenate([2 * srcp, 2 * srcp + 1]).reshape(
        NC * NS * G, CH)
    dstp = dstp.reshape(NS * G, CH)

    zf = jnp.zeros((CH, DC), jnp.float32)
    zc = jnp.zeros((CH, 16), jnp.float32)
    ones_h = jnp.ones((CH, 16), jnp.float32)

    wl1t = Wl1.T
    wr1t = Wr1.T
    wl2t = Wl2.T
    wr2t = Wr2.T
    wl3t = Wl3.T
    wr3t = Wr3.T
    bl1r = bl1.reshape(1, DH1)
    bl2r = bl2.reshape(1, D)
    bl3r = bl3.reshape(1, D)

    pc1 = _sc_counts(dstp, zc, ones_h)
    pf1 = _sc_agg(x.reshape(2 * N, DC), srcp2, dstp, zf)
    h1, f2, inv = _tc1(pf1.reshape(NC, N_ACC, DC), pc1.reshape(NC, N_ACC, 16),
                       x, wl1t, bl1r, wr1t, wl2t)

    pf2 = _sc_agg(f2.reshape(2 * N, DC), srcp2, dstp, zf)
    h2, f3 = _tc2(pf2.reshape(NC, N_ACC, DC), inv, h1, bl2r, wr2t, wl3t)

    pf3 = _sc_agg(f3.reshape(2 * N, DC), srcp2, dstp, zf)
    out = _tc3(pf3.reshape(NC, N_ACC, DC), inv, h1, h2, bl3r, wr3t)
    return out
